# Initial kernel scaffold; baseline (speedup 1.0000x reference)
import jax
import jax.numpy as jnp
from jax.experimental import pallas as pl


def kernel(seq_feats, seq_logits, segment_ids, W_attn, b_attn, doc_label_mask):
    raise NotImplementedError("write your pallas kernel here")


# speedup vs baseline: 4.4024x; 4.4024x over previous
import jax
import jax.numpy as jnp
from jax import lax
from jax.experimental import pallas as pl

N_SEQS = 16384
N_DOCS = 512
HIDDEN = 768
N_CLASSES = 1000
BR = 256
G = N_SEQS // BR


def _body(feats_ref, logits_ref, ids_ref, w_ref, b_ref, mask_ref,
          out_ref, num_ref, den_ref):
    i = pl.program_id(0)

    @pl.when(i == 0)
    def _init():
        num_ref[...] = jnp.zeros_like(num_ref)
        den_ref[...] = jnp.zeros_like(den_ref)

    feats = feats_ref[...]
    scores = lax.dot_general(feats, w_ref[...], (((1,), (0,)), ((), ())),
                             preferred_element_type=jnp.float32)
    s = scores[:, 0] + b_ref[0, 0]
    ex = jnp.exp(s)
    ids = ids_ref[0, 0, :]
    onehot = ids[:, None] == lax.broadcasted_iota(jnp.int32, (BR, N_DOCS), 1)
    m = jnp.where(onehot, ex[:, None], 0.0)
    den_ref[...] += m.sum(axis=0)[None, :]
    num_ref[...] += lax.dot_general(m, logits_ref[...], (((0,), (0,)), ((), ())),
                                    preferred_element_type=jnp.float32)

    @pl.when(i == G - 1)
    def _fin():
        den = den_ref[0, :]
        den = jnp.where(den == 0.0, 1.0, den)
        out_ref[...] = num_ref[...] / den[:, None] + (mask_ref[0, :] - 1.0) * 1e10


def kernel(seq_feats, seq_logits, segment_ids, W_attn, b_attn, doc_label_mask):
    ids3 = segment_ids.astype(jnp.int32).reshape(G, 1, BR)
    b2 = b_attn.reshape(1, 1)
    mask2 = doc_label_mask.reshape(1, N_CLASSES)
    out, _, _ = pl.pallas_call(
        _body,
        grid=(G,),
        in_specs=[
            pl.BlockSpec((BR, HIDDEN), lambda i: (i, 0)),
            pl.BlockSpec((BR, N_CLASSES), lambda i: (i, 0)),
            pl.BlockSpec((1, 1, BR), lambda i: (i, 0, 0)),
            pl.BlockSpec((HIDDEN, 1), lambda i: (0, 0)),
            pl.BlockSpec((1, 1), lambda i: (0, 0)),
            pl.BlockSpec((1, N_CLASSES), lambda i: (0, 0)),
        ],
        out_specs=[
            pl.BlockSpec((N_DOCS, N_CLASSES), lambda i: (0, 0)),
            pl.BlockSpec((N_DOCS, N_CLASSES), lambda i: (0, 0)),
            pl.BlockSpec((1, N_DOCS), lambda i: (0, 0)),
        ],
        out_shape=[
            jax.ShapeDtypeStruct((N_DOCS, N_CLASSES), jnp.float32),
            jax.ShapeDtypeStruct((N_DOCS, N_CLASSES), jnp.float32),
            jax.ShapeDtypeStruct((1, N_DOCS), jnp.float32),
        ],
    )(seq_feats, seq_logits, ids3, W_attn, b2, mask2)
    return out
